# Initial kernel scaffold; baseline (speedup 1.0000x reference)
import jax
import jax.numpy as jnp
from jax.experimental import pallas as pl


def kernel(pooled_embs):
    raise NotImplementedError("write your pallas kernel here")


# speedup vs baseline: 4.0644x; 4.0644x over previous
import jax
import jax.numpy as jnp
from jax.experimental import pallas as pl

_GROUP = 64
_NGROUPS = 26
_WIDTH = _GROUP * _NGROUPS
_PERM = [25 - j for j in range(_NGROUPS)]
_ROWS_PER_BLOCK = 512


def _body(in_ref, out_ref):
    x = in_ref[...]
    parts = [x[:, _GROUP * p:_GROUP * (p + 1)] for p in _PERM]
    out_ref[...] = jnp.concatenate(parts, axis=1)


def kernel(pooled_embs):
    batch, width = pooled_embs.shape
    r = _ROWS_PER_BLOCK
    grid = (batch // r,)
    return pl.pallas_call(
        _body,
        grid=grid,
        in_specs=[pl.BlockSpec((r, width), lambda i: (i, 0))],
        out_specs=pl.BlockSpec((r, width), lambda i: (i, 0)),
        out_shape=jax.ShapeDtypeStruct((batch, width), pooled_embs.dtype),
    )(pooled_embs)
